# baseline (device time: 1484659 ns/iter reference)
import jax
import jax.numpy as jnp
from jax import lax
from jax.experimental import pallas as pl
from jax.experimental.pallas import tpu as pltpu

N_DEV = 32


def kernel(x, w_mat, scale_x, scale_w):
    m_global, k_loc = x.shape
    _, n = w_mat.shape
    m_chunk = m_global // N_DEV

    def body(x_ref, w_ref, sx_ref, sw_ref, out_ref,
             send_ref, recv_ref, send_sems, recv_sems, credit_sem):
        my = lax.axis_index("i")
        left = lax.rem(my + N_DEV - 1, N_DEV)
        right = lax.rem(my + 1, N_DEV)

        barrier_sem = pltpu.get_barrier_semaphore()
        pl.semaphore_signal(barrier_sem, inc=1, device_id=(left,),
                            device_id_type=pl.DeviceIdType.MESH)
        pl.semaphore_signal(barrier_sem, inc=1, device_id=(right,),
                            device_id_type=pl.DeviceIdType.MESH)
        pl.semaphore_wait(barrier_sem, 2)

        def partial(c):
            xs = x_ref[pl.ds(c * m_chunk, m_chunk), :]
            return lax.dot_general(
                xs, w_ref[...],
                dimension_numbers=(((1,), (0,)), ((), ())),
                preferred_element_type=jnp.int32,
            )

        for s in range(N_DEV - 1):
            slot = s % 2
            c = lax.rem(my - (s + 1) + 2 * N_DEV, N_DEV)
            p = partial(c)
            if s == 0:
                send_ref[slot] = p
            else:
                send_ref[slot] = recv_ref[(s - 1) % 2] + p
            if 1 <= s <= N_DEV - 3:
                pl.semaphore_signal(credit_sem, inc=1, device_id=(left,),
                                    device_id_type=pl.DeviceIdType.MESH)
            if s >= 2:
                pl.semaphore_wait(credit_sem, 1)
            rdma = pltpu.make_async_remote_copy(
                src_ref=send_ref.at[slot],
                dst_ref=recv_ref.at[slot],
                send_sem=send_sems.at[slot],
                recv_sem=recv_sems.at[slot],
                device_id=(right,),
                device_id_type=pl.DeviceIdType.MESH,
            )
            rdma.start()
            rdma.wait()

        acc = recv_ref[(N_DEV - 2) % 2] + partial(my)
        scale = sx_ref[0] * sw_ref[0]
        out_ref[...] = acc.astype(jnp.float32) * scale

    return pl.pallas_call(
        body,
        out_shape=jax.ShapeDtypeStruct((m_chunk, n), jnp.float32),
        in_specs=[
            pl.BlockSpec(memory_space=pltpu.VMEM),
            pl.BlockSpec(memory_space=pltpu.VMEM),
            pl.BlockSpec(memory_space=pltpu.VMEM),
            pl.BlockSpec(memory_space=pltpu.VMEM),
        ],
        out_specs=pl.BlockSpec(memory_space=pltpu.VMEM),
        scratch_shapes=[
            pltpu.VMEM((2, m_chunk, n), jnp.int32),
            pltpu.VMEM((2, m_chunk, n), jnp.int32),
            pltpu.SemaphoreType.DMA((2,)),
            pltpu.SemaphoreType.DMA((2,)),
            pltpu.SemaphoreType.REGULAR,
        ],
        compiler_params=pltpu.CompilerParams(collective_id=0),
    )(x, w_mat, scale_x, scale_w)


# device time: 790938 ns/iter; 1.8771x vs baseline; 1.8771x over previous
import jax
import jax.numpy as jnp
from jax import lax
from jax.experimental import pallas as pl
from jax.experimental.pallas import tpu as pltpu

N_DEV = 32


def _mesh_coords():
    coords = []
    for z in range(4):
        for y in range(4):
            xs = (1, 0) if y % 2 else (0, 1)
            coords.extend((x, y, z) for x in xs)
    return coords


def _hamiltonian_ring():
    path = []
    for z in range(4):
        ys = range(4) if z % 2 == 0 else range(3, -1, -1)
        path.extend((y, z) for y in ys)
    cycle = [(0, y, z) for (y, z) in path] + [(1, y, z) for (y, z) in reversed(path)]
    mesh_index = {c: m for m, c in enumerate(_mesh_coords())}
    return [mesh_index[c] for c in cycle]


def kernel(x, w_mat, scale_x, scale_w):
    m_global, k_loc = x.shape
    _, n = w_mat.shape
    m_chunk = m_global // N_DEV
    hn = n // 2

    ring = _hamiltonian_ring()
    ring_arr = jnp.asarray(ring, jnp.int32)
    ringpos = jnp.asarray(
        [ring.index(m) for m in range(N_DEV)], jnp.int32)

    my = lax.axis_index("i")
    r = ringpos[my]
    steps = jnp.arange(N_DEV - 1, dtype=jnp.int32)
    sched_plus = ring_arr[(r - 1 - steps) % N_DEV]
    sched_minus = ring_arr[(r + 1 + steps) % N_DEV]
    nbrs = jnp.stack([ring_arr[(r + 1) % N_DEV],
                      ring_arr[(r - 1) % N_DEV]])

    def body(x_ref, w_ref, sx_ref, sw_ref, sp_ref, sm_ref, nbr_ref, out_ref,
             send_a, recv_a, send_b, recv_b,
             ssem_a, rsem_a, ssem_b, rsem_b, credit_a, credit_b):
        succ = nbr_ref[0]
        pred = nbr_ref[1]

        barrier_sem = pltpu.get_barrier_semaphore()
        pl.semaphore_signal(barrier_sem, inc=1, device_id=(succ,),
                            device_id_type=pl.DeviceIdType.MESH)
        pl.semaphore_signal(barrier_sem, inc=1, device_id=(pred,),
                            device_id_type=pl.DeviceIdType.MESH)
        pl.semaphore_wait(barrier_sem, 2)

        def partial(c, col0):
            xs = x_ref[pl.ds(c * m_chunk, m_chunk), :]
            return lax.dot_general(
                xs, w_ref[:, col0:col0 + hn],
                dimension_numbers=(((1,), (0,)), ((), ())),
                preferred_element_type=jnp.int32,
            )

        for s in range(N_DEV - 1):
            slot = s % 2
            pa = partial(sp_ref[s], 0)
            pb = partial(sm_ref[s], hn)
            if s == 0:
                send_a[slot] = pa
                send_b[slot] = pb
            else:
                send_a[slot] = recv_a[(s - 1) % 2] + pa
                send_b[slot] = recv_b[(s - 1) % 2] + pb
            if 1 <= s <= N_DEV - 3:
                pl.semaphore_signal(credit_a, inc=1, device_id=(pred,),
                                    device_id_type=pl.DeviceIdType.MESH)
                pl.semaphore_signal(credit_b, inc=1, device_id=(succ,),
                                    device_id_type=pl.DeviceIdType.MESH)
            if s >= 2:
                pl.semaphore_wait(credit_a, 1)
                pl.semaphore_wait(credit_b, 1)
            rdma_a = pltpu.make_async_remote_copy(
                src_ref=send_a.at[slot], dst_ref=recv_a.at[slot],
                send_sem=ssem_a.at[slot], recv_sem=rsem_a.at[slot],
                device_id=(succ,), device_id_type=pl.DeviceIdType.MESH,
            )
            rdma_b = pltpu.make_async_remote_copy(
                src_ref=send_b.at[slot], dst_ref=recv_b.at[slot],
                send_sem=ssem_b.at[slot], recv_sem=rsem_b.at[slot],
                device_id=(pred,), device_id_type=pl.DeviceIdType.MESH,
            )
            rdma_a.start()
            rdma_b.start()
            rdma_a.wait()
            rdma_b.wait()

        my_chunk = lax.axis_index("i")
        acc_a = recv_a[(N_DEV - 2) % 2] + partial(my_chunk, 0)
        acc_b = recv_b[(N_DEV - 2) % 2] + partial(my_chunk, hn)
        scale = sx_ref[0] * sw_ref[0]
        out_ref[:, :hn] = acc_a.astype(jnp.float32) * scale
        out_ref[:, hn:] = acc_b.astype(jnp.float32) * scale

    vmem = pl.BlockSpec(memory_space=pltpu.VMEM)
    smem = pl.BlockSpec(memory_space=pltpu.SMEM)
    return pl.pallas_call(
        body,
        out_shape=jax.ShapeDtypeStruct((m_chunk, n), jnp.float32),
        in_specs=[vmem, vmem, vmem, vmem, smem, smem, smem],
        out_specs=vmem,
        scratch_shapes=[
            pltpu.VMEM((2, m_chunk, hn), jnp.int32),
            pltpu.VMEM((2, m_chunk, hn), jnp.int32),
            pltpu.VMEM((2, m_chunk, hn), jnp.int32),
            pltpu.VMEM((2, m_chunk, hn), jnp.int32),
            pltpu.SemaphoreType.DMA((2,)),
            pltpu.SemaphoreType.DMA((2,)),
            pltpu.SemaphoreType.DMA((2,)),
            pltpu.SemaphoreType.DMA((2,)),
            pltpu.SemaphoreType.REGULAR,
            pltpu.SemaphoreType.REGULAR,
        ],
        compiler_params=pltpu.CompilerParams(collective_id=0),
    )(x, w_mat, scale_x, scale_w, sched_plus, sched_minus, nbrs)


# device time: 780029 ns/iter; 1.9033x vs baseline; 1.0140x over previous
import jax
import jax.numpy as jnp
from jax import lax
from jax.experimental import pallas as pl
from jax.experimental.pallas import tpu as pltpu

N_DEV = 32


def _mesh_coords():
    coords = []
    for z in range(4):
        for y in range(4):
            xs = (1, 0) if y % 2 else (0, 1)
            coords.extend((x, y, z) for x in xs)
    return coords


def _hamiltonian_ring():
    path = []
    for z in range(4):
        ys = range(4) if z % 2 == 0 else range(3, -1, -1)
        path.extend((y, z) for y in ys)
    cycle = [(0, y, z) for (y, z) in path] + [(1, y, z) for (y, z) in reversed(path)]
    mesh_index = {c: m for m, c in enumerate(_mesh_coords())}
    return [mesh_index[c] for c in cycle]


def kernel(x, w_mat, scale_x, scale_w):
    m_global, k_loc = x.shape
    _, n = w_mat.shape
    m_chunk = m_global // N_DEV
    hn = n // 2

    ring = _hamiltonian_ring()
    ring_arr = jnp.asarray(ring, jnp.int32)
    ringpos = jnp.asarray(
        [ring.index(m) for m in range(N_DEV)], jnp.int32)

    my = lax.axis_index("i")
    r = ringpos[my]
    steps = jnp.arange(N_DEV - 1, dtype=jnp.int32)
    sched_plus = ring_arr[(r - 1 - steps) % N_DEV]
    sched_minus = ring_arr[(r + 1 + steps) % N_DEV]
    nbrs = jnp.stack([ring_arr[(r + 1) % N_DEV],
                      ring_arr[(r - 1) % N_DEV]])

    def body(x_ref, w_ref, sx_ref, sw_ref, sp_ref, sm_ref, nbr_ref, out_ref,
             send_a, recv_a, send_b, recv_b,
             ssem_a, rsem_a, ssem_b, rsem_b, credit_a, credit_b):
        succ = nbr_ref[0]
        pred = nbr_ref[1]

        barrier_sem = pltpu.get_barrier_semaphore()
        pl.semaphore_signal(barrier_sem, inc=1, device_id=(succ,),
                            device_id_type=pl.DeviceIdType.MESH)
        pl.semaphore_signal(barrier_sem, inc=1, device_id=(pred,),
                            device_id_type=pl.DeviceIdType.MESH)
        pl.semaphore_wait(barrier_sem, 2)

        def partial(c, col0):
            xs = x_ref[pl.ds(c * m_chunk, m_chunk), :]
            return lax.dot_general(
                xs, w_ref[:, col0:col0 + hn],
                dimension_numbers=(((1,), (0,)), ((), ())),
                preferred_element_type=jnp.int32,
            )

        send_a[0] = partial(sp_ref[0], 0)
        send_b[0] = partial(sm_ref[0], hn)

        my_chunk = lax.axis_index("i")
        scale = sx_ref[0] * sw_ref[0]
        rd_a_prev = rd_b_prev = None
        for s in range(N_DEV - 1):
            slot = s % 2
            if s >= 2:
                pl.semaphore_wait(credit_a, 1)
                pl.semaphore_wait(credit_b, 1)
            rd_a = pltpu.make_async_remote_copy(
                src_ref=send_a.at[slot], dst_ref=recv_a.at[slot],
                send_sem=ssem_a.at[slot], recv_sem=rsem_a.at[slot],
                device_id=(succ,), device_id_type=pl.DeviceIdType.MESH,
            )
            rd_b = pltpu.make_async_remote_copy(
                src_ref=send_b.at[slot], dst_ref=recv_b.at[slot],
                send_sem=ssem_b.at[slot], recv_sem=rsem_b.at[slot],
                device_id=(pred,), device_id_type=pl.DeviceIdType.MESH,
            )
            rd_a.start()
            rd_b.start()
            if s < N_DEV - 2:
                nxt_a = partial(sp_ref[s + 1], 0)
                nxt_b = partial(sm_ref[s + 1], hn)
            else:
                nxt_a = partial(my_chunk, 0)
                nxt_b = partial(my_chunk, hn)
            if s >= 1:
                rd_a_prev.wait_send()
                rd_b_prev.wait_send()
            rd_a.wait_recv()
            rd_b.wait_recv()
            if s < N_DEV - 2:
                send_a[1 - slot] = recv_a[slot] + nxt_a
                send_b[1 - slot] = recv_b[slot] + nxt_b
                if s <= N_DEV - 4:
                    pl.semaphore_signal(credit_a, inc=1, device_id=(pred,),
                                        device_id_type=pl.DeviceIdType.MESH)
                    pl.semaphore_signal(credit_b, inc=1, device_id=(succ,),
                                        device_id_type=pl.DeviceIdType.MESH)
            else:
                acc_a = recv_a[slot] + nxt_a
                acc_b = recv_b[slot] + nxt_b
                out_ref[:, :hn] = acc_a.astype(jnp.float32) * scale
                out_ref[:, hn:] = acc_b.astype(jnp.float32) * scale
            rd_a_prev, rd_b_prev = rd_a, rd_b

        rd_a_prev.wait_send()
        rd_b_prev.wait_send()

    vmem = pl.BlockSpec(memory_space=pltpu.VMEM)
    smem = pl.BlockSpec(memory_space=pltpu.SMEM)
    return pl.pallas_call(
        body,
        out_shape=jax.ShapeDtypeStruct((m_chunk, n), jnp.float32),
        in_specs=[vmem, vmem, vmem, vmem, smem, smem, smem],
        out_specs=vmem,
        scratch_shapes=[
            pltpu.VMEM((2, m_chunk, hn), jnp.int32),
            pltpu.VMEM((2, m_chunk, hn), jnp.int32),
            pltpu.VMEM((2, m_chunk, hn), jnp.int32),
            pltpu.VMEM((2, m_chunk, hn), jnp.int32),
            pltpu.SemaphoreType.DMA((2,)),
            pltpu.SemaphoreType.DMA((2,)),
            pltpu.SemaphoreType.DMA((2,)),
            pltpu.SemaphoreType.DMA((2,)),
            pltpu.SemaphoreType.REGULAR,
            pltpu.SemaphoreType.REGULAR,
        ],
        compiler_params=pltpu.CompilerParams(collective_id=0),
    )(x, w_mat, scale_x, scale_w, sched_plus, sched_minus, nbrs)
